# baseline (speedup 1.0000x reference)
import functools

import jax
import jax.numpy as jnp
from jax import lax
from jax.experimental import pallas as pl
from jax.experimental.pallas import tpu as pltpu
from jax.experimental.pallas import tpu_sc as plsc

FEAT = 64
BATCH = 16384
NUM_ROWS = 1000000
NC = 2
NS = 16
L = 16
NW = NC * NS
BPW = BATCH // NW
NGROUPS = BPW // L
W = 1536
FBLK = 128
TAIL_START = (NUM_ROWS // FBLK) * FBLK
NTAIL = NUM_ROWS - TAIL_START
MAX_WS = ((NUM_ROWS - W) // 128) * 128
DONE = 0x40000000


def _make_kernel():
    mesh = plsc.VectorSubcoreMesh(core_axis_name="c", subcore_axis_name="s")

    @functools.partial(
        pl.kernel,
        mesh=mesh,
        compiler_params=pltpu.CompilerParams(needs_layout_passes=False),
        out_type=jax.ShapeDtypeStruct((NW, L), jnp.float32),
        scratch_types=[
            pltpu.VMEM((BPW,), jnp.int32),
            pltpu.VMEM((FEAT, FBLK), jnp.float32),
            pltpu.VMEM((8, 8, W), jnp.float32),
            pltpu.VMEM((FEAT, NTAIL), jnp.float32),
            pltpu.VMEM((L,), jnp.float32),
            pltpu.SMEM((1,), jnp.int32),
            pltpu.SemaphoreType.DMA,
            pltpu.SemaphoreType.DMA,
        ],
    )
    def center_loss_partial(feat_hbm, lab_hbm, cent_hbm, tail_hbm, out_hbm,
                            labs_v, feat_v, win_v, tail_v, out_v, ws_smem,
                            sem0, sem1):
        wid = lax.axis_index("s") * NC + lax.axis_index("c")
        base = wid * BPW
        pltpu.sync_copy(lab_hbm.at[pl.ds(base, BPW)], labs_v)
        pltpu.sync_copy(tail_hbm, tail_v)
        out_v[...] = jnp.zeros((L,), jnp.float32)
        ws_smem[0] = jnp.int32(-(2 * W))

        def sq_contrib(cols, pmask, table, goff):
            accs = [jnp.zeros((L,), jnp.float32) for _ in range(4)]
            for f in range(FEAT):
                cv = table(f, cols)
                d = feat_v[f, pl.ds(goff, L)] - cv
                accs[f % 4] = accs[f % 4] + d * d
            tot = (accs[0] + accs[1]) + (accs[2] + accs[3])
            return jnp.where(pmask, tot, jnp.float32(0.0))

        def group_body(g, _):
            @pl.when(lax.rem(g, 8) == 0)
            def _load_feat():
                pltpu.sync_copy(
                    feat_hbm.at[:, pl.ds(base + FBLK * (g // 8), FBLK)],
                    feat_v)

            goff = lax.rem(g, 8) * L
            labs16 = labs_v[pl.ds(g * L, L)]

            tmask = labs16 >= TAIL_START
            rem0 = jnp.where(tmask, DONE, labs16)

            @pl.when(jnp.max(labs16) >= TAIL_START)
            def _tail():
                cols_t = jnp.clip(labs16 - TAIL_START, 0, NTAIL - 1)

                def tail_at(f, cc):
                    return plsc.load_gather(
                        tail_v, [jnp.full((L,), f, jnp.int32), cc])

                out_v[...] += sq_contrib(cols_t, tmask, tail_at, goff)

            def cond(rem):
                return jnp.min(rem) < DONE

            def body(rem):
                m = jnp.min(rem)

                @pl.when(
                    jnp.logical_or(m < ws_smem[0], m >= ws_smem[0] + W))
                def _slide():
                    ws_smem[0] = jnp.minimum(
                        (m // 128) * 128, jnp.int32(MAX_WS))
                    start = pl.multiple_of(ws_smem[0], 128)
                    sems = (sem0, sem1)
                    descs = [
                        pltpu.async_copy(
                            cent_hbm.at[tr, :, pl.ds(start, W)],
                            win_v.at[tr], sems[tr % 2])
                        for tr in range(8)
                    ]
                    for d in descs:
                        d.wait()

                ws = ws_smem[0]
                pmask = jnp.logical_and(rem >= ws, rem < ws + W)
                cols = jnp.clip(rem - ws, 0, W - 1)

                def win_at(f, cc):
                    return plsc.load_gather(
                        win_v, [jnp.full((L,), f // 8, jnp.int32),
                                jnp.full((L,), f % 8, jnp.int32), cc])

                out_v[...] += sq_contrib(cols, pmask, win_at, goff)
                return jnp.where(pmask, DONE, rem)

            lax.while_loop(cond, body, rem0)
            return 0

        lax.fori_loop(0, NGROUPS, group_body, 0)
        inv = jnp.float32(1.0 / (BATCH * FEAT))
        out_v[...] = out_v[...] * inv
        pltpu.sync_copy(out_v, out_hbm.at[wid])

    return center_loss_partial


_center_loss_call = _make_kernel()


def kernel(features, labels, centers):
    lab = labels.astype(jnp.int32)
    order = jnp.argsort(lab)
    lab_sorted = lab[order]
    feat_t = features[order].T
    cent_t = centers.T.reshape(8, 8, NUM_ROWS)
    tail_t = centers[TAIL_START:].T
    partial = _center_loss_call(feat_t, lab_sorted, cent_t, tail_t)
    return jnp.sum(partial)


# speedup vs baseline: 1.0775x; 1.0775x over previous
import functools

import jax
import jax.numpy as jnp
from jax import lax
from jax.experimental import pallas as pl
from jax.experimental.pallas import tpu as pltpu
from jax.experimental.pallas import tpu_sc as plsc

FEAT = 64
BATCH = 16384
NUM_ROWS = 1000000
NC = 2
NS = 16
L = 16
NW = NC * NS
BPW = BATCH // NW
NGROUPS = BPW // L
W = 1536
FBLK = 128
TAIL_START = (NUM_ROWS // FBLK) * FBLK
NTAIL = NUM_ROWS - TAIL_START
MAX_WS = ((NUM_ROWS - W) // 128) * 128
DONE = 0x40000000


def _make_kernel():
    mesh = plsc.VectorSubcoreMesh(core_axis_name="c", subcore_axis_name="s")

    @functools.partial(
        pl.kernel,
        mesh=mesh,
        compiler_params=pltpu.CompilerParams(needs_layout_passes=False),
        out_type=jax.ShapeDtypeStruct((NW, L), jnp.float32),
        scratch_types=[
            pltpu.VMEM((BPW,), jnp.int32),
            pltpu.VMEM((FEAT, FBLK), jnp.float32),
            pltpu.VMEM((FEAT, W), jnp.float32),
            pltpu.VMEM((FEAT, NTAIL), jnp.float32),
            pltpu.VMEM((L,), jnp.float32),
            pltpu.SMEM((1,), jnp.int32),
            pltpu.SemaphoreType.DMA,
            pltpu.SemaphoreType.DMA,
        ],
    )
    def center_loss_partial(feat_hbm, lab_hbm, cent_hbm, tail_hbm, out_hbm,
                            labs_v, feat_v, win_v, tail_v, out_v, ws_smem,
                            sem0, sem1):
        wid = lax.axis_index("s") * NC + lax.axis_index("c")
        base = wid * BPW
        pltpu.sync_copy(lab_hbm.at[pl.ds(base, BPW)], labs_v)
        pltpu.sync_copy(tail_hbm, tail_v)
        out_v[...] = jnp.zeros((L,), jnp.float32)
        ws_smem[0] = jnp.int32(-(2 * W))

        def sq_contrib(cols, pmask, table, goff):
            accs = [jnp.zeros((L,), jnp.float32) for _ in range(4)]
            for f in range(FEAT):
                cv = table(f, cols)
                d = feat_v[f, pl.ds(goff, L)] - cv
                accs[f % 4] = accs[f % 4] + d * d
            tot = (accs[0] + accs[1]) + (accs[2] + accs[3])
            return jnp.where(pmask, tot, jnp.float32(0.0))

        def group_body(g, _):
            @pl.when(lax.rem(g, 8) == 0)
            def _load_feat():
                pltpu.sync_copy(
                    feat_hbm.at[:, pl.ds(base + FBLK * (g // 8), FBLK)],
                    feat_v)

            goff = lax.rem(g, 8) * L
            labs16 = labs_v[pl.ds(g * L, L)]

            tmask = labs16 >= TAIL_START
            rem0 = jnp.where(tmask, DONE, labs16)

            @pl.when(jnp.max(labs16) >= TAIL_START)
            def _tail():
                cols_t = jnp.clip(labs16 - TAIL_START, 0, NTAIL - 1)

                def tail_at(f, cc):
                    return plsc.load_gather(
                        tail_v, [jnp.full((L,), f, jnp.int32), cc])

                out_v[...] += sq_contrib(cols_t, tmask, tail_at, goff)

            def cond(rem):
                return jnp.min(rem) < DONE

            def body(rem):
                m = jnp.min(rem)

                @pl.when(
                    jnp.logical_or(m < ws_smem[0], m >= ws_smem[0] + W))
                def _slide():
                    ws_smem[0] = jnp.minimum(
                        (m // 128) * 128, jnp.int32(MAX_WS))
                    start = pl.multiple_of(ws_smem[0], 128)
                    pltpu.sync_copy(cent_hbm.at[:, pl.ds(start, W)], win_v)

                ws = ws_smem[0]
                pmask = jnp.logical_and(rem >= ws, rem < ws + W)
                cols = jnp.clip(rem - ws, 0, W - 1)

                def win_at(f, cc):
                    return plsc.load_gather(
                        win_v, [jnp.full((L,), f, jnp.int32), cc])

                out_v[...] += sq_contrib(cols, pmask, win_at, goff)
                return jnp.where(pmask, DONE, rem)

            lax.while_loop(cond, body, rem0)
            return 0

        lax.fori_loop(0, NGROUPS, group_body, 0)
        inv = jnp.float32(1.0 / (BATCH * FEAT))
        out_v[...] = out_v[...] * inv
        pltpu.sync_copy(out_v, out_hbm.at[wid])

    return center_loss_partial


_center_loss_call = _make_kernel()


def kernel(features, labels, centers):
    lab = labels.astype(jnp.int32)
    order = jnp.argsort(lab)
    lab_sorted = lab[order]
    feat_t = features[order].T
    cent_t = centers.T
    tail_t = centers[TAIL_START:].T
    partial = _center_loss_call(feat_t, lab_sorted, cent_t, tail_t)
    return jnp.sum(partial)
